# speedup vs baseline: 6.0245x; 6.0245x over previous
import jax
import jax.numpy as jnp
from jax import lax
from jax.experimental import pallas as pl

HIDDEN_DIM = 64
VOCAB_SIZE = 64
MEMORY_SLOTS = 4
SEQ_LEN = 48
B = 4096

VOCAB_PAD = 128


def _fused_kernel(idx_ref, embed_ref, W1_ref, b1_ref, W2_ref, b2_ref, out_ref):
    T = idx_ref.shape[0]
    idx = idx_ref[...]
    iota = lax.broadcasted_iota(jnp.int32, (T, VOCAB_PAD), 1)

    q_oh = (idx[:, 0:1] == iota).astype(jnp.float32)
    m_oh = (idx[:, 1:2] == iota).astype(jnp.float32)
    for j in range(2, 1 + MEMORY_SLOTS):
        m_oh += (idx[:, j:j + 1] == iota).astype(jnp.float32)

    embed = embed_ref[...]
    e1 = jnp.dot(embed, W1_ref[0:HIDDEN_DIM, :],
                 preferred_element_type=jnp.float32)
    e2 = jnp.dot(embed, W1_ref[HIDDEN_DIM:2 * HIDDEN_DIM, :],
                 preferred_element_type=jnp.float32) * (1.0 / MEMORY_SLOTS)

    h = jnp.dot(q_oh, e1, preferred_element_type=jnp.float32)
    h += jnp.dot(m_oh, e2, preferred_element_type=jnp.float32)
    h += b1_ref[...]
    h = jnp.maximum(h, 0.0)
    out_ref[...] = jnp.dot(h, W2_ref[...],
                           preferred_element_type=jnp.float32) + b2_ref[...]


def kernel(seqs, query_tok, embed, W1, b1, W2, b2):
    mem_idx = seqs[:, SEQ_LEN - 1 - MEMORY_SLOTS: SEQ_LEN - 1]
    idx = jnp.concatenate(
        [query_tok[:, None].astype(jnp.int32), mem_idx.astype(jnp.int32),
         jnp.full((B, 3), -1, dtype=jnp.int32)], axis=1)

    embed_p = jnp.zeros((VOCAB_PAD, HIDDEN_DIM), jnp.float32).at[
        0:VOCAB_SIZE + 2].set(embed)

    TILE = 1024
    grid = (B // TILE,)
    return pl.pallas_call(
        _fused_kernel,
        grid=grid,
        in_specs=[
            pl.BlockSpec((TILE, 8), lambda i: (i, 0)),
            pl.BlockSpec((VOCAB_PAD, HIDDEN_DIM), lambda i: (0, 0)),
            pl.BlockSpec((2 * HIDDEN_DIM, HIDDEN_DIM), lambda i: (0, 0)),
            pl.BlockSpec((HIDDEN_DIM,), lambda i: (0,)),
            pl.BlockSpec((HIDDEN_DIM, VOCAB_SIZE), lambda i: (0, 0)),
            pl.BlockSpec((VOCAB_SIZE,), lambda i: (0,)),
        ],
        out_specs=pl.BlockSpec((TILE, VOCAB_SIZE), lambda i: (i, 0)),
        out_shape=jax.ShapeDtypeStruct((B, VOCAB_SIZE), jnp.float32),
    )(idx, embed_p, W1, b1, W2, b2)
